# Initial kernel scaffold; baseline (speedup 1.0000x reference)
import jax
import jax.numpy as jnp
from jax.experimental import pallas as pl


def kernel(x, edge_index, W1, b1, W2, b2):
    raise NotImplementedError("write your pallas kernel here")


# speedup vs baseline: 19.1860x; 19.1860x over previous
import functools

import jax
import jax.numpy as jnp
from jax import lax
from jax.experimental import pallas as pl
from jax.experimental.pallas import tpu as pltpu, tpu_sc as plsc

N = 10000
E = 320000
D = 128
NC = 2
NS = 16
NW = NC * NS
EW = E // NW
K = 125
NCHUNK = EW // K
NP = 10240
RPT = NP // NS

_mesh = plsc.VectorSubcoreMesh(core_axis_name="c", subcore_axis_name="s")


@functools.partial(
    pl.kernel,
    out_type=jax.ShapeDtypeStruct((NC, NP), jnp.float32),
    mesh=_mesh,
    scratch_types=[
        pltpu.VMEM_SHARED((NP,), jnp.float32),
        pltpu.VMEM((NCHUNK, K), jnp.int32),
        pltpu.VMEM((NP // NS,), jnp.float32),
        pltpu.VMEM((128,), jnp.float32),
    ],
)
def _sc_degree(dst_hbm, out_hbm, dacc, dst_v, zb, ones_v):
    c = lax.axis_index("c")
    s = lax.axis_index("s")
    w = c * NS + s

    def zstep(i, carry):
        zb[pl.ds(i * 16, 16)] = jnp.zeros((16,), jnp.float32)
        return carry

    lax.fori_loop(0, (NP // NS) // 16, zstep, 0)
    for i in range(8):
        ones_v[pl.ds(i * 16, 16)] = jnp.ones((16,), jnp.float32)
    pltpu.sync_copy(zb, dacc.at[pl.ds(s * (NP // NS), NP // NS)])
    plsc.subcore_barrier()

    pltpu.sync_copy(dst_hbm.at[pl.ds(w * NCHUNK, NCHUNK)], dst_v)

    def step(j, carry):
        pltpu.sync_copy(ones_v.at[pl.ds(0, K)], dacc.at[dst_v.at[j]], add=True)
        return carry

    lax.fori_loop(0, NCHUNK, step, 0)
    plsc.subcore_barrier()

    pltpu.sync_copy(dacc.at[pl.ds(s * (NP // NS), NP // NS)], zb)
    pltpu.sync_copy(zb, out_hbm.at[c, pl.ds(s * (NP // NS), NP // NS)])


@functools.partial(
    pl.kernel,
    out_type=jax.ShapeDtypeStruct((NC, NP, D), jnp.float32),
    mesh=_mesh,
    scratch_types=[
        pltpu.VMEM_SHARED((NP, D), jnp.float32),
        pltpu.VMEM((NCHUNK, K), jnp.int32),
        pltpu.VMEM((NCHUNK, K), jnp.int32),
        pltpu.VMEM((K, D), jnp.float32),
        pltpu.VMEM((64, D), jnp.float32),
    ],
)
def _sc_edge_agg(h_hbm, src_hbm, dst_hbm, out_hbm, acc, src_v, dst_v, buf, zbuf):
    c = lax.axis_index("c")
    s = lax.axis_index("s")
    w = c * NS + s

    def zrow(i, carry):
        for k2 in range(D // 16):
            zbuf[i, pl.ds(k2 * 16, 16)] = jnp.zeros((16,), jnp.float32)
        return carry

    lax.fori_loop(0, 64, zrow, 0)
    for k in range(RPT // 64):
        pltpu.sync_copy(zbuf, acc.at[pl.ds(s * RPT + k * 64, 64)])
    plsc.subcore_barrier()

    pltpu.sync_copy(src_hbm.at[pl.ds(w * NCHUNK, NCHUNK)], src_v)
    pltpu.sync_copy(dst_hbm.at[pl.ds(w * NCHUNK, NCHUNK)], dst_v)

    def step(j, carry):
        pltpu.sync_copy(h_hbm.at[src_v.at[j]], buf)
        pltpu.sync_copy(buf, acc.at[dst_v.at[j]], add=True)
        return carry

    lax.fori_loop(0, NCHUNK, step, 0)
    plsc.subcore_barrier()

    for k in range(RPT // 64):
        rows = s * RPT + k * 64
        pltpu.sync_copy(acc.at[pl.ds(rows, 64)], zbuf)
        pltpu.sync_copy(zbuf, out_hbm.at[c, pl.ds(rows, 64)])


_BLK = 1000
_GRID = N // _BLK


def _tc_first_body(d_ref, x_ref, w_ref, h_ref, r_ref):
    deg = d_ref[:, 0] + d_ref[:, 1]
    r = lax.rsqrt(jnp.maximum(deg, 1.0))
    h = jnp.dot(x_ref[...], w_ref[...], preferred_element_type=jnp.float32)
    h_ref[...] = h * r[:, None]
    r_ref[...] = r[:, None]


def _tc_first(degp, x, W1):
    return pl.pallas_call(
        _tc_first_body,
        grid=(_GRID,),
        in_specs=[
            pl.BlockSpec((_BLK, NC), lambda i: (i, 0)),
            pl.BlockSpec((_BLK, D), lambda i: (i, 0)),
            pl.BlockSpec((D, D), lambda i: (0, 0)),
        ],
        out_specs=[
            pl.BlockSpec((_BLK, D), lambda i: (i, 0)),
            pl.BlockSpec((_BLK, 1), lambda i: (i, 0)),
        ],
        out_shape=[
            jax.ShapeDtypeStruct((N, D), jnp.float32),
            jax.ShapeDtypeStruct((N, 1), jnp.float32),
        ],
    )(degp, x, W1)


def _tc_mid_body(p_ref, r_ref, b_ref, w_ref, o_ref):
    r = r_ref[...]
    a = (p_ref[0] + p_ref[1]) * r + b_ref[...]
    o_ref[...] = jnp.dot(a, w_ref[...], preferred_element_type=jnp.float32) * r


def _tc_mid(p, r, b1, W2):
    return pl.pallas_call(
        _tc_mid_body,
        grid=(_GRID,),
        in_specs=[
            pl.BlockSpec((NC, _BLK, D), lambda i: (0, i, 0)),
            pl.BlockSpec((_BLK, 1), lambda i: (i, 0)),
            pl.BlockSpec((1, D), lambda i: (0, 0)),
            pl.BlockSpec((D, D), lambda i: (0, 0)),
        ],
        out_specs=pl.BlockSpec((_BLK, D), lambda i: (i, 0)),
        out_shape=jax.ShapeDtypeStruct((N, D), jnp.float32),
    )(p, r, b1, W2)


def _tc_final_body(q_ref, r_ref, b_ref, o_ref):
    o_ref[...] = (q_ref[0] + q_ref[1]) * r_ref[...] + b_ref[...]


def _tc_final(q, r, b2):
    return pl.pallas_call(
        _tc_final_body,
        grid=(_GRID,),
        in_specs=[
            pl.BlockSpec((NC, _BLK, D), lambda i: (0, i, 0)),
            pl.BlockSpec((_BLK, 1), lambda i: (i, 0)),
            pl.BlockSpec((1, D), lambda i: (0, 0)),
        ],
        out_specs=pl.BlockSpec((_BLK, D), lambda i: (i, 0)),
        out_shape=jax.ShapeDtypeStruct((N, D), jnp.float32),
    )(q, r, b2)


def kernel(x, edge_index, W1, b1, W2, b2):
    src = edge_index[0].astype(jnp.int32).reshape(NW * NCHUNK, K)
    dst = edge_index[1].astype(jnp.int32).reshape(NW * NCHUNK, K)
    degp = _sc_degree(dst)[:, :N].T
    h1p, r = _tc_first(degp, x, W1)
    p = _sc_edge_agg(h1p, src, dst)
    h2p = _tc_mid(p, r, b1.reshape(1, D), W2)
    q = _sc_edge_agg(h2p, src, dst)
    return _tc_final(q, r, b2.reshape(1, D))
